# Initial kernel scaffold; baseline (speedup 1.0000x reference)
import jax
import jax.numpy as jnp
from jax.experimental import pallas as pl


def kernel(in_feats, in_coords, W, bias):
    raise NotImplementedError("write your pallas kernel here")


# speedup vs baseline: 56.2945x; 56.2945x over previous
import functools
import math

import jax
import jax.numpy as jnp
from jax import lax
from jax.experimental import pallas as pl
from jax.experimental.pallas import tpu as pltpu
from jax.experimental.pallas import tpu_sc as plsc

N = 64
H = 8
CIN = 768
COUT = 768
B = 8192
LAYER_NUM = 5

BM = 256
MT = B // BM
G = MT + N - 1


def _sc_row_gather(table, idx):
    R, D = table.shape
    info = plsc.get_sparse_core_info()
    NC, NS = info.num_cores, info.num_subcores
    NW = NC * NS
    rows_per_w = R // NW
    CH = 128
    n_chunks = rows_per_w // CH

    mesh = plsc.VectorSubcoreMesh(core_axis_name="c", subcore_axis_name="s")

    @functools.partial(
        pl.kernel,
        mesh=mesh,
        out_type=jax.ShapeDtypeStruct((R, D), table.dtype),
        scratch_types=[
            pltpu.VMEM((CH,), jnp.int32),
            pltpu.VMEM((CH, D), table.dtype),
            pltpu.SemaphoreType.DMA,
        ],
    )
    def k(table_hbm, idx_hbm, out_hbm, idx_v, rows_v, sem):
        wid = lax.axis_index("s") * NC + lax.axis_index("c")
        for c in range(n_chunks):
            base = wid * rows_per_w + c * CH
            pltpu.sync_copy(idx_hbm.at[pl.ds(base, CH)], idx_v)
            pltpu.async_copy(table_hbm.at[idx_v], rows_v, sem).wait()
            pltpu.sync_copy(rows_v, out_hbm.at[pl.ds(base, CH)])

    return k(table, idx)


def _gmm_body(grp_s, mt_s, lo_s, hi_s, x_ref, w_ref, b_ref, o_ref):
    j = pl.program_id(0)
    mt = mt_s[j]
    lo = lo_s[j]
    hi = hi_s[j]
    rows = mt * BM + lax.broadcasted_iota(jnp.int32, (BM, 1), 0)
    mask = (rows >= lo) & (rows < hi)

    x = x_ref[...]
    w = w_ref[0]
    part = lax.dot_general(
        x, w, (((1,), (1,)), ((), ())), preferred_element_type=jnp.float32
    )
    part = jnp.where(mask, part, 0.0)

    prev_mt = mt_s[jnp.maximum(j - 1, 0)]
    next_mt = mt_s[jnp.minimum(j + 1, G - 1)]
    is_first = (j == 0) | (mt != prev_mt)
    is_last = (j == G - 1) | (mt != next_mt)

    prev = jnp.where(is_first, jnp.zeros_like(part), o_ref[...])
    acc = prev + part
    final = acc + b_ref[...]
    final = jnp.where(final >= 0, final, 0.2 * final)
    o_ref[...] = jnp.where(is_last, final, acc)


def _gmm(x_sorted, W, bias2d, grp, mt, lo, hi):
    grid_spec = pltpu.PrefetchScalarGridSpec(
        num_scalar_prefetch=4,
        grid=(G,),
        in_specs=[
            pl.BlockSpec((BM, CIN), lambda j, g, m, l, h: (m[j], 0)),
            pl.BlockSpec((1, COUT, CIN), lambda j, g, m, l, h: (g[j], 0, 0)),
            pl.BlockSpec((1, COUT), lambda j, g, m, l, h: (0, 0)),
        ],
        out_specs=pl.BlockSpec((BM, COUT), lambda j, g, m, l, h: (m[j], 0)),
    )
    return pl.pallas_call(
        _gmm_body,
        grid_spec=grid_spec,
        out_shape=jax.ShapeDtypeStruct((B, COUT), jnp.float32),
        compiler_params=pltpu.CompilerParams(
            dimension_semantics=("arbitrary",),
        ),
    )(grp, mt, lo, hi, x_sorted, W, bias2d)


def _routing(in_coords):
    A = 2 ** (LAYER_NUM - 1)
    b = 0.5
    aff = in_coords * A + b
    xg = jnp.floor(aff[:, 0]).astype(jnp.int32) % H
    yg = jnp.floor(aff[:, 1]).astype(jnp.int32) % H
    tile = H * xg + yg

    perm = jnp.argsort(tile).astype(jnp.int32)
    inv_perm = (
        jnp.zeros((B,), jnp.int32).at[perm].set(jnp.arange(B, dtype=jnp.int32))
    )

    sizes = jnp.bincount(tile, length=N).astype(jnp.int32)
    ends = jnp.cumsum(sizes)
    starts = ends - sizes
    blocks = jnp.where(sizes > 0, (ends - 1) // BM - starts // BM + 1, 0)
    u = jnp.cumsum(blocks)
    total = u[-1]

    j = jnp.arange(G, dtype=jnp.int32)
    g = jnp.searchsorted(u, j, side="right").astype(jnp.int32)
    valid = j < total
    gc = jnp.minimum(g, N - 1)
    k_in_g = j - (u[gc] - blocks[gc])
    mt = jnp.where(valid, starts[gc] // BM + k_in_g, MT - 1).astype(jnp.int32)
    lo = jnp.where(valid, jnp.maximum(starts[gc], mt * BM), B).astype(jnp.int32)
    hi = jnp.where(
        valid, jnp.minimum(ends[gc], (mt + 1) * BM), B
    ).astype(jnp.int32)
    grp = jnp.where(valid, gc, N - 1).astype(jnp.int32)
    return perm, inv_perm, grp, mt, lo, hi


def kernel(in_feats, in_coords, W, bias):
    perm, inv_perm, grp, mt, lo, hi = _routing(in_coords)
    x_sorted = _sc_row_gather(in_feats, perm)
    bias2d = bias.reshape(1, COUT)
    out_sorted = _gmm(x_sorted, W, bias2d, grp, mt, lo, hi)
    return _sc_row_gather(out_sorted, inv_perm)
